# Initial kernel scaffold; baseline (speedup 1.0000x reference)
import jax
import jax.numpy as jnp
from jax.experimental import pallas as pl


def kernel(node_feat, edge_feat, src_idx, dst_idx, e2e_dst, e2e_src, W_n, W_e, W_r):
    raise NotImplementedError("write your pallas kernel here")


# speedup vs baseline: 2.6905x; 2.6905x over previous
import functools

import jax
import jax.numpy as jnp
from jax import lax
from jax.experimental import pallas as pl
from jax.experimental.pallas import tpu as pltpu
from jax.experimental.pallas import tpu_sc as plsc

N = 10000
E = 320000
K = 1280000
D = 128
DE = 16
OUT = 128
STEPS = 3

NC = 2
NS = 16
NW = NC * NS
L = 16

B = 128
WIN = 10000
NWIN = E // WIN
NPASS = NWIN // NC
WROWS = WIN + 16
STRIPE = 632

RROWS = N + 16

EPW = E // NW
NFULL = EPW // B
TAIL = EPW - NFULL * B

_mesh = plsc.VectorSubcoreMesh(core_axis_name="c", subcore_axis_name="s")
_lanes = None


def _lane_extract(vec, p):
  lanes = lax.iota(jnp.int32, L)
  return jnp.sum(jnp.where(lanes == p, vec, 0))


def _gather_body(tbl, idx_hbm, out_hbm, idxv, idxv_t, rows, rows_t, sem):
  c = lax.axis_index("c")
  t = lax.axis_index("s")
  w = t * NC + c
  base = w * EPW

  def body(b, _):
    off = pl.multiple_of(base + b * B, 8)
    pltpu.sync_copy(idx_hbm.at[pl.ds(off, B)], idxv)
    pltpu.async_copy(tbl.at[idxv], rows, sem).wait()
    pltpu.sync_copy(rows, out_hbm.at[pl.ds(off, B)])
    return 0

  lax.fori_loop(0, NFULL, body, 0)
  off = pl.multiple_of(base + NFULL * B, 8)
  pltpu.sync_copy(idx_hbm.at[pl.ds(off, TAIL)], idxv_t)
  pltpu.async_copy(tbl.at[idxv_t], rows_t, sem).wait()
  pltpu.sync_copy(rows_t, out_hbm.at[pl.ds(off, TAIL)])


_gather_fn = pl.kernel(
    _gather_body,
    out_type=jax.ShapeDtypeStruct((E, D), jnp.float32),
    mesh=_mesh,
    scratch_types=[
        pltpu.VMEM((B,), jnp.int32),
        pltpu.VMEM((TAIL,), jnp.int32),
        pltpu.VMEM((B, D), jnp.float32),
        pltpu.VMEM((TAIL, D), jnp.float32),
        pltpu.SemaphoreType.DMA,
    ],
)


def _zero_zbuf(zbuf):
  zero = jnp.zeros((L,), jnp.float32)

  def zb(j, _):
    for l in range(D // L):
      zbuf[j, pl.ds(l * L, L)] = zero
    return 0

  lax.fori_loop(0, B, zb, 0)


def _zero_stripe(zbuf, shared, row0, nrows):
  for j in range(nrows // B):
    pltpu.sync_copy(zbuf, shared.at[pl.ds(pl.multiple_of(row0 + j * B, 8), B)])
  rem = nrows % B
  if rem:
    pltpu.sync_copy(
        zbuf.at[pl.ds(0, rem)],
        shared.at[pl.ds(pl.multiple_of(row0 + (nrows // B) * B, 8), rem)])


def _agg_body(msg, ssrc, sdst, ranges, agg,
              rgv, idxv, dstv, locv, rows, zbuf, sem, shared):
  c = lax.axis_index("c")
  t = lax.axis_index("s")
  lanes = lax.iota(jnp.int32, L)
  pltpu.sync_copy(ranges.at[c, t], rgv)
  _zero_zbuf(zbuf)
  sv = rgv[0]
  ev = rgv[1]
  zrow = jnp.where(t == NS - 1, WROWS - STRIPE, t * STRIPE)
  orow = jnp.where(t == NS - 1, WIN - STRIPE, t * STRIPE)

  for p in range(NPASS):
    wbase = (c * NPASS + p) * WIN
    ts = sv[p]
    te = ev[p]
    sa = ts & jnp.int32(~7)
    nb = (te - sa + (B - 1)) // B

    _zero_stripe(zbuf, shared, zrow, STRIPE)
    plsc.subcore_barrier()

    tsv = jnp.full((L,), ts, jnp.int32)
    tev = jnp.full((L,), te, jnp.int32)
    wbv = jnp.full((L,), wbase, jnp.int32)

    def bat(b, _):
      off = pl.multiple_of(sa + b * B, 8)
      pltpu.sync_copy(ssrc.at[pl.ds(off, B)], idxv)
      pltpu.sync_copy(sdst.at[pl.ds(off, B)], dstv)
      gat = pltpu.async_copy(msg.at[idxv], rows, sem)
      offv = jnp.full((L,), off, jnp.int32)
      for ch in range(B // L):
        dv = dstv[pl.ds(ch * L, L)]
        pos = offv + (lanes + (ch * L))
        ok = (pos >= tsv) & (pos < tev)
        locv[pl.ds(ch * L, L)] = jnp.where(ok, dv - wbv, WIN)
      gat.wait()
      pltpu.sync_copy(rows, shared.at[locv], add=True)
      return 0

    lax.fori_loop(0, nb, bat, 0)
    plsc.subcore_barrier()
    pltpu.sync_copy(
        shared.at[pl.ds(pl.multiple_of(orow, 8), STRIPE)],
        agg.at[pl.ds(pl.multiple_of(wbase + orow, 8), STRIPE)])
    plsc.subcore_barrier()


_agg_fn = pl.kernel(
    _agg_body,
    out_type=jax.ShapeDtypeStruct((E, D), jnp.float32),
    mesh=_mesh,
    scratch_types=[
        pltpu.VMEM((2, L), jnp.int32),
        pltpu.VMEM((B,), jnp.int32),
        pltpu.VMEM((B,), jnp.int32),
        pltpu.VMEM((B,), jnp.int32),
        pltpu.VMEM((B, D), jnp.float32),
        pltpu.VMEM((B, D), jnp.float32),
        pltpu.SemaphoreType.DMA,
        pltpu.VMEM_SHARED((WROWS, D), jnp.float32),
    ],
)


def _red_body(msg, dsti, part, dstv, dstv_t, rows, rows_t, zbuf, sem, shared):
  c = lax.axis_index("c")
  t = lax.axis_index("s")
  _zero_zbuf(zbuf)
  zrow = jnp.where(t == NS - 1, RROWS - STRIPE, t * STRIPE)
  _zero_stripe(zbuf, shared, zrow, STRIPE)
  plsc.subcore_barrier()

  ebase = (c * NS + t) * EPW

  def bat(b, _):
    off = pl.multiple_of(ebase + b * B, 8)
    pltpu.sync_copy(dsti.at[pl.ds(off, B)], dstv)
    pltpu.sync_copy(msg.at[pl.ds(off, B)], rows)
    pltpu.sync_copy(rows, shared.at[dstv], add=True)
    return 0

  lax.fori_loop(0, NFULL, bat, 0)
  off = pl.multiple_of(ebase + NFULL * B, 8)
  pltpu.sync_copy(dsti.at[pl.ds(off, TAIL)], dstv_t)
  pltpu.sync_copy(msg.at[pl.ds(off, TAIL)], rows_t)
  pltpu.sync_copy(rows_t, shared.at[dstv_t], add=True)

  plsc.subcore_barrier()
  orow = pl.multiple_of(
      jnp.where(t == NS - 1, N - STRIPE, t * STRIPE), 8)
  pltpu.sync_copy(shared.at[pl.ds(orow, STRIPE)],
                  part.at[c, pl.ds(orow, STRIPE)])


_red_fn = pl.kernel(
    _red_body,
    out_type=jax.ShapeDtypeStruct((NC, N, D), jnp.float32),
    mesh=_mesh,
    scratch_types=[
        pltpu.VMEM((B,), jnp.int32),
        pltpu.VMEM((TAIL,), jnp.int32),
        pltpu.VMEM((B, D), jnp.float32),
        pltpu.VMEM((TAIL, D), jnp.float32),
        pltpu.VMEM((B, D), jnp.float32),
        pltpu.SemaphoreType.DMA,
        pltpu.VMEM_SHARED((RROWS, D), jnp.float32),
    ],
)


def _proj_body(x_ref, w_ref, o_ref):
  o_ref[...] = jnp.dot(x_ref[...], w_ref[...],
                       preferred_element_type=jnp.float32)


_NBLK = 2000
_proj_fn = pl.pallas_call(
    _proj_body,
    grid=(N // _NBLK,),
    in_specs=[
        pl.BlockSpec((_NBLK, D), lambda i: (i, 0)),
        pl.BlockSpec((D, OUT), lambda i: (0, 0)),
    ],
    out_specs=pl.BlockSpec((_NBLK, OUT), lambda i: (i, 0)),
    out_shape=jax.ShapeDtypeStruct((N, OUT), jnp.float32),
)


def _inmsg_body(g_ref, ef_ref, we_ref, im_ref, m_ref):
  x = g_ref[...] + jnp.dot(ef_ref[...], we_ref[...],
                           preferred_element_type=jnp.float32)
  im_ref[...] = x
  m_ref[...] = jnp.maximum(x, 0.0)


_EBLK = 512
_inmsg_fn = pl.pallas_call(
    _inmsg_body,
    grid=(E // _EBLK,),
    in_specs=[
        pl.BlockSpec((_EBLK, OUT), lambda i: (i, 0)),
        pl.BlockSpec((_EBLK, DE), lambda i: (i, 0)),
        pl.BlockSpec((DE, OUT), lambda i: (0, 0)),
    ],
    out_specs=[
        pl.BlockSpec((_EBLK, OUT), lambda i: (i, 0)),
        pl.BlockSpec((_EBLK, OUT), lambda i: (i, 0)),
    ],
    out_shape=[
        jax.ShapeDtypeStruct((E, OUT), jnp.float32),
        jax.ShapeDtypeStruct((E, OUT), jnp.float32),
    ],
)


def _step_body(im_ref, agg_ref, wr_ref, m_ref):
  m_ref[...] = jnp.maximum(
      im_ref[...] + jnp.dot(agg_ref[...], wr_ref[...],
                            preferred_element_type=jnp.float32),
      0.0)


_step_fn = pl.pallas_call(
    _step_body,
    grid=(E // _EBLK,),
    in_specs=[
        pl.BlockSpec((_EBLK, OUT), lambda i: (i, 0)),
        pl.BlockSpec((_EBLK, OUT), lambda i: (i, 0)),
        pl.BlockSpec((OUT, OUT), lambda i: (0, 0)),
    ],
    out_specs=pl.BlockSpec((_EBLK, OUT), lambda i: (i, 0)),
    out_shape=jax.ShapeDtypeStruct((E, OUT), jnp.float32),
)


def _psum_body(p_ref, o_ref):
  o_ref[...] = p_ref[0] + p_ref[1]


_psum_fn = pl.pallas_call(
    _psum_body,
    grid=(N // _NBLK,),
    in_specs=[pl.BlockSpec((NC, _NBLK, OUT), lambda i: (0, i, 0))],
    out_specs=pl.BlockSpec((_NBLK, OUT), lambda i: (i, 0)),
    out_shape=jax.ShapeDtypeStruct((N, OUT), jnp.float32),
)


def kernel(node_feat, edge_feat, src_idx, dst_idx, e2e_dst, e2e_src,
           W_n, W_e, W_r):
  sdst, ssrc = lax.sort_key_val(e2e_dst, e2e_src)
  ssrc_p = jnp.concatenate([ssrc, jnp.zeros((B,), jnp.int32)])
  sdst_p = jnp.concatenate([sdst, jnp.full((B,), E, jnp.int32)])
  wb = jnp.searchsorted(
      sdst, jnp.arange(NWIN + 1, dtype=jnp.int32) * WIN).astype(jnp.int32)
  ws, we = wb[:-1], wb[1:]
  tt = jnp.arange(NS, dtype=jnp.int32)[:, None]
  span = (we - ws)[None, :]
  ts = ws[None, :] + span * tt // NS
  te = ws[None, :] + span * (tt + 1) // NS
  ts = ts.reshape(NS, NC, NPASS).transpose(1, 0, 2)
  te = te.reshape(NS, NC, NPASS).transpose(1, 0, 2)
  pad = [(0, 0), (0, 0), (0, L - NPASS)]
  ranges = jnp.stack(
      [jnp.pad(ts, pad), jnp.pad(te, pad)], axis=2).astype(jnp.int32)

  proj = _proj_fn(node_feat, W_n)
  gathered = _gather_fn(proj, src_idx)
  input_message, message = _inmsg_fn(gathered, edge_feat, W_e)
  for _ in range(STEPS):
    agg = _agg_fn(message, ssrc_p, sdst_p, ranges)
    message = _step_fn(input_message, agg, W_r)
  parts = _red_fn(message, dst_idx)
  return _psum_fn(parts)


# speedup vs baseline: 3.0159x; 1.1209x over previous
import functools

import jax
import jax.numpy as jnp
from jax import lax
from jax.experimental import pallas as pl
from jax.experimental.pallas import tpu as pltpu
from jax.experimental.pallas import tpu_sc as plsc

N = 10000
E = 320000
K = 1280000
D = 128
DE = 16
OUT = 128
STEPS = 3

NC = 2
NS = 16
NW = NC * NS
L = 16

B = 128
SB = 256
WIN = 4096
NWIN = (E + WIN - 1) // WIN
NPASS = 40
NWIN = NC * NPASS
AGG_PAD = NWIN * WIN
WROWS = WIN + 16
OSTRIPE = WIN // NS
STRIPE = 264
PADP = 2048

RROWS = N + 16
RSTRIPE = 632

EPW = E // NW
NFULL = EPW // B
TAIL = EPW - NFULL * B

_mesh = plsc.VectorSubcoreMesh(core_axis_name="c", subcore_axis_name="s")
_lanes = None


def _lane_extract(vec, p):
  lanes = lax.iota(jnp.int32, L)
  return jnp.sum(jnp.where(lanes == p, vec, 0))


def _gather_body(tbl, idx_hbm, out_hbm, idxv, idxv_t, rows, rows_t, sem):
  c = lax.axis_index("c")
  t = lax.axis_index("s")
  w = t * NC + c
  base = w * EPW

  def body(b, _):
    off = pl.multiple_of(base + b * B, 8)
    pltpu.sync_copy(idx_hbm.at[pl.ds(off, B)], idxv)
    pltpu.async_copy(tbl.at[idxv], rows, sem).wait()
    pltpu.sync_copy(rows, out_hbm.at[pl.ds(off, B)])
    return 0

  lax.fori_loop(0, NFULL, body, 0)
  off = pl.multiple_of(base + NFULL * B, 8)
  pltpu.sync_copy(idx_hbm.at[pl.ds(off, TAIL)], idxv_t)
  pltpu.async_copy(tbl.at[idxv_t], rows_t, sem).wait()
  pltpu.sync_copy(rows_t, out_hbm.at[pl.ds(off, TAIL)])


_gather_fn = pl.kernel(
    _gather_body,
    out_type=jax.ShapeDtypeStruct((E, D), jnp.float32),
    mesh=_mesh,
    scratch_types=[
        pltpu.VMEM((B,), jnp.int32),
        pltpu.VMEM((TAIL,), jnp.int32),
        pltpu.VMEM((B, D), jnp.float32),
        pltpu.VMEM((TAIL, D), jnp.float32),
        pltpu.SemaphoreType.DMA,
    ],
)


def _zero_zbuf(zbuf):
  zero = jnp.zeros((L,), jnp.float32)

  def zb(j, _):
    for l in range(D // L):
      zbuf[j, pl.ds(l * L, L)] = zero
    return 0

  lax.fori_loop(0, B, zb, 0)


def _zero_stripe(zbuf, shared, row0, nrows):
  for j in range(nrows // B):
    pltpu.sync_copy(zbuf, shared.at[pl.ds(pl.multiple_of(row0 + j * B, 8), B)])
  rem = nrows % B
  if rem:
    pltpu.sync_copy(
        zbuf.at[pl.ds(0, rem)],
        shared.at[pl.ds(pl.multiple_of(row0 + (nrows // B) * B, 8), rem)])


HCH = 2048
SBPC = HCH // SB


def _agg_body(msg, ssrc, sdst, ranges, agg,
              rgv, ibuf, dbuf, locva, locvb, rows, zbuf,
              sem_g, sem_s, shared):
  c = lax.axis_index("c")
  t = lax.axis_index("s")
  lanes = lax.iota(jnp.int32, L)
  _zero_zbuf(zbuf)
  zrow = jnp.where(t == NS - 1, WROWS - STRIPE, t * STRIPE)
  orow = t * OSTRIPE

  def _issue_gathers(rb, s):
    pltpu.async_copy(
        msg.at[ibuf.at[pl.ds(s * SB, B)]], rows.at[rb, pl.ds(0, B)], sem_g)
    pltpu.async_copy(
        msg.at[ibuf.at[pl.ds(s * SB + B, B)]], rows.at[rb, pl.ds(B, B)],
        sem_g)

  def _wait_gathers(rb, s):
    pltpu.make_async_copy(
        msg.at[ibuf.at[pl.ds(s * SB, B)]], rows.at[rb, pl.ds(0, B)],
        sem_g).wait()
    pltpu.make_async_copy(
        msg.at[ibuf.at[pl.ds(s * SB + B, B)]], rows.at[rb, pl.ds(B, B)],
        sem_g).wait()

  def pass_body(p, _):
    wbase = (c * NPASS + p) * WIN
    pltpu.sync_copy(ranges.at[(c * NS + t) * NPASS + p], rgv)
    rv = rgv[...]
    ts = rv[0]
    te = rv[1]
    sa = ts & jnp.int32(~7)
    nb = (te - sa + (SB - 1)) // SB
    nch = (nb + (SBPC - 1)) // SBPC

    _zero_stripe(zbuf, shared, zrow, STRIPE)
    plsc.subcore_barrier()

    tsv = jnp.full((L,), ts, jnp.int32)
    tev = jnp.full((L,), te, jnp.int32)
    wbv = jnp.full((L,), wbase, jnp.int32)

    def chunk_body(kc, _):
      coff = pl.multiple_of(sa + kc * HCH, 8)
      pltpu.sync_copy(ssrc.at[pl.ds(coff, HCH)], ibuf)
      pltpu.sync_copy(sdst.at[pl.ds(coff, HCH)], dbuf)
      nsb = jnp.minimum(nb - kc * SBPC, SBPC)

      @pl.when(nsb > 0)
      def _prologue():
        _issue_gathers(0, 0)

      def sb_body(s, _):
        rb = s % 2
        rb1 = (s + 1) % 2
        off = coff + s * SB
        _wait_gathers(rb, s)
        @pl.when(s + 1 < nsb)
        def _next():
          _issue_gathers(rb1, s + 1)
        offv = jnp.full((L,), off, jnp.int32)
        for half, lv in ((0, locva), (1, locvb)):
          for ch in range(B // L):
            dv = dbuf[pl.ds(s * SB + half * B + ch * L, L)]
            pos = offv + (lanes + (half * B + ch * L))
            ok = (pos >= tsv) & (pos < tev)
            lv[pl.ds(ch * L, L)] = jnp.where(ok, dv - wbv, WIN)
        s0 = pltpu.async_copy(
            rows.at[rb, pl.ds(0, B)], shared.at[locva], sem_s, add=True)
        s1 = pltpu.async_copy(
            rows.at[rb, pl.ds(B, B)], shared.at[locvb], sem_s, add=True)
        s0.wait()
        s1.wait()
        return 0

      lax.fori_loop(0, nsb, sb_body, 0)
      return 0

    lax.fori_loop(0, nch, chunk_body, 0)
    plsc.subcore_barrier()
    pltpu.sync_copy(
        shared.at[pl.ds(pl.multiple_of(orow, 8), OSTRIPE)],
        agg.at[pl.ds(pl.multiple_of(wbase + orow, 8), OSTRIPE)])
    plsc.subcore_barrier()
    return 0

  lax.fori_loop(0, NPASS, pass_body, 0)


_agg_fn = pl.kernel(
    _agg_body,
    out_type=jax.ShapeDtypeStruct((AGG_PAD, D), jnp.float32),
    mesh=_mesh,
    scratch_types=[
        pltpu.VMEM((L,), jnp.int32),
        pltpu.VMEM((HCH,), jnp.int32),
        pltpu.VMEM((HCH,), jnp.int32),
        pltpu.VMEM((B,), jnp.int32),
        pltpu.VMEM((B,), jnp.int32),
        pltpu.VMEM((2, SB, D), jnp.float32),
        pltpu.VMEM((B, D), jnp.float32),
        pltpu.SemaphoreType.DMA,
        pltpu.SemaphoreType.DMA,
        pltpu.VMEM_SHARED((WROWS, D), jnp.float32),
    ],
)


def _red_body(msg, dsti, part, dstv, dstv_t, rows, rows_t, zbuf, sem, shared):
  c = lax.axis_index("c")
  t = lax.axis_index("s")
  _zero_zbuf(zbuf)
  zrow = jnp.where(t == NS - 1, RROWS - RSTRIPE, t * RSTRIPE)
  _zero_stripe(zbuf, shared, zrow, RSTRIPE)
  plsc.subcore_barrier()

  ebase = (c * NS + t) * EPW

  def bat(b, _):
    off = pl.multiple_of(ebase + b * B, 8)
    pltpu.sync_copy(dsti.at[pl.ds(off, B)], dstv)
    pltpu.sync_copy(msg.at[pl.ds(off, B)], rows)
    pltpu.sync_copy(rows, shared.at[dstv], add=True)
    return 0

  lax.fori_loop(0, NFULL, bat, 0)
  off = pl.multiple_of(ebase + NFULL * B, 8)
  pltpu.sync_copy(dsti.at[pl.ds(off, TAIL)], dstv_t)
  pltpu.sync_copy(msg.at[pl.ds(off, TAIL)], rows_t)
  pltpu.sync_copy(rows_t, shared.at[dstv_t], add=True)

  plsc.subcore_barrier()
  orow = pl.multiple_of(
      jnp.where(t == NS - 1, N - RSTRIPE, t * RSTRIPE), 8)
  pltpu.sync_copy(shared.at[pl.ds(orow, RSTRIPE)],
                  part.at[c, pl.ds(orow, RSTRIPE)])


_red_fn = pl.kernel(
    _red_body,
    out_type=jax.ShapeDtypeStruct((NC, N, D), jnp.float32),
    mesh=_mesh,
    scratch_types=[
        pltpu.VMEM((B,), jnp.int32),
        pltpu.VMEM((TAIL,), jnp.int32),
        pltpu.VMEM((B, D), jnp.float32),
        pltpu.VMEM((TAIL, D), jnp.float32),
        pltpu.VMEM((B, D), jnp.float32),
        pltpu.SemaphoreType.DMA,
        pltpu.VMEM_SHARED((RROWS, D), jnp.float32),
    ],
)


def _proj_body(x_ref, w_ref, o_ref):
  o_ref[...] = jnp.dot(x_ref[...], w_ref[...],
                       preferred_element_type=jnp.float32)


_NBLK = 2000
_proj_fn = pl.pallas_call(
    _proj_body,
    grid=(N // _NBLK,),
    in_specs=[
        pl.BlockSpec((_NBLK, D), lambda i: (i, 0)),
        pl.BlockSpec((D, OUT), lambda i: (0, 0)),
    ],
    out_specs=pl.BlockSpec((_NBLK, OUT), lambda i: (i, 0)),
    out_shape=jax.ShapeDtypeStruct((N, OUT), jnp.float32),
)


def _inmsg_body(g_ref, ef_ref, we_ref, im_ref, m_ref):
  x = g_ref[...] + jnp.dot(ef_ref[...], we_ref[...],
                           preferred_element_type=jnp.float32)
  im_ref[...] = x
  m_ref[...] = jnp.maximum(x, 0.0)


_EBLK = 512
_inmsg_fn = pl.pallas_call(
    _inmsg_body,
    grid=(E // _EBLK,),
    in_specs=[
        pl.BlockSpec((_EBLK, OUT), lambda i: (i, 0)),
        pl.BlockSpec((_EBLK, DE), lambda i: (i, 0)),
        pl.BlockSpec((DE, OUT), lambda i: (0, 0)),
    ],
    out_specs=[
        pl.BlockSpec((_EBLK, OUT), lambda i: (i, 0)),
        pl.BlockSpec((_EBLK, OUT), lambda i: (i, 0)),
    ],
    out_shape=[
        jax.ShapeDtypeStruct((E, OUT), jnp.float32),
        jax.ShapeDtypeStruct((E, OUT), jnp.float32),
    ],
)


def _step_body(im_ref, agg_ref, wr_ref, m_ref):
  m_ref[...] = jnp.maximum(
      im_ref[...] + jnp.dot(agg_ref[...], wr_ref[...],
                            preferred_element_type=jnp.float32),
      0.0)


_step_fn = pl.pallas_call(
    _step_body,
    grid=(E // _EBLK,),
    in_specs=[
        pl.BlockSpec((_EBLK, OUT), lambda i: (i, 0)),
        pl.BlockSpec((_EBLK, OUT), lambda i: (i, 0)),
        pl.BlockSpec((OUT, OUT), lambda i: (0, 0)),
    ],
    out_specs=pl.BlockSpec((_EBLK, OUT), lambda i: (i, 0)),
    out_shape=jax.ShapeDtypeStruct((E, OUT), jnp.float32),
)


def _psum_body(p_ref, o_ref):
  o_ref[...] = p_ref[0] + p_ref[1]


_psum_fn = pl.pallas_call(
    _psum_body,
    grid=(N // _NBLK,),
    in_specs=[pl.BlockSpec((NC, _NBLK, OUT), lambda i: (0, i, 0))],
    out_specs=pl.BlockSpec((_NBLK, OUT), lambda i: (i, 0)),
    out_shape=jax.ShapeDtypeStruct((N, OUT), jnp.float32),
)


def kernel(node_feat, edge_feat, src_idx, dst_idx, e2e_dst, e2e_src,
           W_n, W_e, W_r):
  sdst, ssrc = lax.sort_key_val(e2e_dst, e2e_src)
  ssrc_p = jnp.concatenate([ssrc, jnp.zeros((PADP,), jnp.int32)])
  sdst_p = jnp.concatenate([sdst, jnp.full((PADP,), E, jnp.int32)])
  wb = jnp.searchsorted(
      sdst, jnp.arange(NWIN + 1, dtype=jnp.int32) * WIN).astype(jnp.int32)
  ws, we = wb[:-1], wb[1:]
  tt = jnp.arange(NS, dtype=jnp.int32)[:, None]
  span = (we - ws)[None, :]
  ts = ws[None, :] + span * tt // NS
  te = ws[None, :] + span * (tt + 1) // NS
  ts = ts.reshape(NS, NC, NPASS).transpose(1, 0, 2)
  te = te.reshape(NS, NC, NPASS).transpose(1, 0, 2)
  ranges = jnp.stack(
      [ts, te] + [jnp.zeros_like(ts)] * (L - 2), axis=3).astype(
          jnp.int32).reshape(NC * NS * NPASS, L)

  proj = _proj_fn(node_feat, W_n)
  gathered = _gather_fn(proj, src_idx)
  input_message, message = _inmsg_fn(gathered, edge_feat, W_e)
  for _ in range(STEPS):
    agg = _agg_fn(message, ssrc_p, sdst_p, ranges)
    message = _step_fn(input_message, agg, W_r)
  parts = _red_fn(message, dst_idx)
  return _psum_fn(parts)
